# baseline (speedup 1.0000x reference)
import jax
import jax.numpy as jnp
from jax.experimental import pallas as pl

_BS = 256


def _add_kernel(x_ref, t_ref, o_ref):
    o_ref[...] = x_ref[...] + t_ref[...][None, :, :]


def kernel(x, pos_table):
    B, S, E = x.shape
    return pl.pallas_call(
        _add_kernel,
        grid=(S // _BS,),
        in_specs=[
            pl.BlockSpec((B, _BS, E), lambda j: (0, j, 0)),
            pl.BlockSpec((_BS, E), lambda j: (j, 0)),
        ],
        out_specs=pl.BlockSpec((B, _BS, E), lambda j: (0, j, 0)),
        out_shape=jax.ShapeDtypeStruct((B, S, E), x.dtype),
    )(x, pos_table)


# speedup vs baseline: 1.0299x; 1.0299x over previous
import jax
import jax.numpy as jnp
from jax.experimental import pallas as pl

_BS = 1024


def _add_kernel(x_ref, t_ref, o_ref):
    o_ref[...] = x_ref[...] + t_ref[...][None, :, :]


def kernel(x, pos_table):
    B, S, E = x.shape
    return pl.pallas_call(
        _add_kernel,
        grid=(S // _BS,),
        in_specs=[
            pl.BlockSpec((B, _BS, E), lambda j: (0, j, 0)),
            pl.BlockSpec((_BS, E), lambda j: (j, 0)),
        ],
        out_specs=pl.BlockSpec((B, _BS, E), lambda j: (0, j, 0)),
        out_shape=jax.ShapeDtypeStruct((B, S, E), x.dtype),
    )(x, pos_table)
